# baseline (speedup 1.0000x reference)
import functools

import jax
import jax.numpy as jnp
from jax import lax
from jax.experimental import pallas as pl
from jax.experimental.pallas import tpu as pltpu
from jax.experimental.pallas import tpu_sc as plsc

N = 10000
E = 160000
EB = 128
NW = 32
NB_N = 256
NBLK = 40
CE = 4608
SLOT = NBLK * CE
SPT = SLOT // NW
SBT = SPT // EB
NS = NBLK * NB_N
M_BLK = 1000
GRID = N // M_BLK

_MESH = plsc.VectorSubcoreMesh(
    core_axis_name="c", subcore_axis_name="s", num_cores=2, num_subcores=16
)


_DEPTH = 6


def _make_gather(width):
    eb = 128 if width <= 256 else 64
    sbt = SPT // eb
    D = _DEPTH

    @functools.partial(
        pl.kernel,
        out_type=jax.ShapeDtypeStruct((SLOT, width // 2), jnp.int32),
        mesh=_MESH,
        scratch_types=[
            pltpu.VMEM((sbt, eb), jnp.int32),
            pltpu.VMEM((D, eb, width // 2), jnp.int32),
            pltpu.SemaphoreType.DMA((D,)),
            pltpu.SemaphoreType.DMA((D,)),
        ],
    )
    def gather(table, srcs, msg, idxb, rows, gsem, wsem):
        c = lax.axis_index("c")
        s = lax.axis_index("s")
        w = s * 2 + c
        pltpu.sync_copy(srcs.at[w], idxb)

        def gstart(k, b):
            pltpu.async_copy(table.at[idxb.at[k]], rows.at[b], gsem.at[b])

        def gwait(k, b):
            pltpu.make_async_copy(table.at[idxb.at[k]], rows.at[b],
                                  gsem.at[b]).wait()

        def wstart(k, b):
            pltpu.async_copy(rows.at[b], msg.at[pl.ds(w * SPT + k * eb, eb)],
                             wsem.at[b])

        def wwait(k, b):
            pltpu.make_async_copy(rows.at[b],
                                  msg.at[pl.ds(w * SPT + k * eb, eb)],
                                  wsem.at[b]).wait()

        for j in range(D - 1):
            gstart(j, j)

        @pl.loop(0, sbt)
        def _(k):
            b = lax.rem(k, D)
            gwait(k, b)
            wstart(k, b)
            kn = k + D - 1

            @pl.when(kn < sbt)
            def _():
                bn = lax.rem(kn, D)

                @pl.when(k >= 1)
                def _():
                    wwait(k - 1, bn)

                gstart(kn, bn)

        for t in range(D):
            kk = sbt - 1 - t
            wwait(kk, lax.rem(kk, D))

    return gather, eb, sbt


_gather256, _EB256, _SBT256 = _make_gather(256)
_gather512, _EB512, _SBT512 = _make_gather(512)


def _segsum(msg, dstl3, emit_deg=False):
    width = msg.shape[1]

    def body(*refs):
        dl_ref, msg_ref = refs[0], refs[1]
        s_ref = refs[2]
        dl = dl_ref[0, 0, :]
        oh = jnp.where(
            lax.broadcasted_iota(jnp.int32, (NB_N, CE), 0) == dl[None, :],
            1.0, 0.0)
        s_ref[...] = jnp.dot(oh.astype(jnp.bfloat16), msg_ref[...],
                             preferred_element_type=jnp.float32)
        if emit_deg:
            refs[3][...] = jnp.broadcast_to(
                jnp.sum(oh, axis=1, keepdims=True), (NB_N, 8))

    in_specs = [
        pl.BlockSpec((1, 1, CE), lambda i: (i, 0, 0)),
        pl.BlockSpec((CE, width), lambda i: (i, 0)),
    ]
    out_shape = [jax.ShapeDtypeStruct((NS, width), jnp.float32)]
    out_specs = [pl.BlockSpec((NB_N, width), lambda i: (i, 0))]
    if emit_deg:
        out_shape.append(jax.ShapeDtypeStruct((NS, 8), jnp.float32))
        out_specs.append(pl.BlockSpec((NB_N, 8), lambda i: (i, 0)))
    return pl.pallas_call(
        body, grid=(NBLK,), in_specs=in_specs,
        out_specs=out_specs, out_shape=out_shape,
    )(dstl3, msg)


def _tc_layer(s_parts, x_parts, Wl_parts, Wr_parts, b, deg8, act,
              scale=None, shift=None, s_pre=True, fuse_W_parts=None,
              emit_bf=False, fuse_bf=False):
    ns, nx = len(s_parts), len(x_parts)
    dout = Wr_parts[0].shape[1]
    nW = len(Wl_parts) if s_pre else 0
    nf = len(fuse_W_parts) if fuse_W_parts is not None else 0
    dnext = fuse_W_parts[0].shape[1] if nf else 0
    has_bn = scale is not None

    def body(*refs):
        i = 0
        s_r = refs[i:i + ns]; i += ns
        x_r = refs[i:i + nx]; i += nx
        wl_r = refs[i:i + nW]; i += nW
        wr_r = refs[i:i + nx]; i += nx
        b_r = refs[i]; i += 1
        if has_bn:
            sc_r = refs[i]; i += 1
            sh_r = refs[i]; i += 1
        fw_r = refs[i:i + nf]; i += nf
        deg_r = refs[i]; i += 1
        out_r = refs[i]; i += 1
        bf_r = None
        if emit_bf:
            bf_r = refs[i]; i += 1
        y_r = refs[i] if nf else None

        invd = 1.0 / jnp.maximum(deg_r[...][:, 0:1], 1.0)
        z = b_r[...]
        if s_pre:
            for k in range(ns):
                z = z + jnp.dot(s_r[k][...] * invd, wl_r[k][...],
                                preferred_element_type=jnp.float32)
        else:
            z = z + s_r[0][...] * invd
        for k in range(nx):
            z = z + jnp.dot(x_r[k][...], wr_r[k][...],
                            preferred_element_type=jnp.float32)

        if act == "relu":
            h = jnp.maximum(z, 0.0)
        elif act == "bnrelu":
            h = jnp.maximum(z, 0.0) * sc_r[...] + sh_r[...]
        elif act == "lsm40":
            zm = jnp.where(lax.broadcasted_iota(jnp.int32, z.shape, 1) >= 40,
                           -1e30, z)
            m = jnp.max(zm, axis=1, keepdims=True)
            lse = m + jnp.log(jnp.sum(jnp.exp(zm - m), axis=1, keepdims=True))
            h = zm - lse
        else:
            h = z

        if act == "lsm40":
            out_r[...] = h[:, :40]
        else:
            out_r[...] = h
        if emit_bf:
            bf_r[...] = h.astype(jnp.bfloat16)
        if nf:
            y = jnp.dot(h[:, 0:256], fw_r[0][...],
                        preferred_element_type=jnp.float32)
            for k in range(1, nf):
                y = y + jnp.dot(h[:, k * 256:(k + 1) * 256], fw_r[k][...],
                                preferred_element_type=jnp.float32)
            y_r[...] = y.astype(jnp.bfloat16) if fuse_bf else y

    def pspec(part):
        a, cb = part
        wpart = min(256, a.shape[1])
        return a, pl.BlockSpec((M_BLK, wpart), lambda i, cb=cb: (i, cb))

    row = lambda shp: pl.BlockSpec((M_BLK, shp), lambda i: (i, 0))
    full = lambda a: pl.BlockSpec(a.shape, lambda i: (0, 0))

    in_arrays, in_specs = [], []
    for p in s_parts:
        a, sp = pspec(p); in_arrays.append(a); in_specs.append(sp)
    for p in x_parts:
        a, sp = pspec(p); in_arrays.append(a); in_specs.append(sp)
    if s_pre:
        for a in Wl_parts:
            in_arrays.append(a); in_specs.append(full(a))
    for a in Wr_parts:
        in_arrays.append(a); in_specs.append(full(a))
    b2 = b.reshape(1, -1)
    in_arrays.append(b2); in_specs.append(full(b2))
    if has_bn:
        for a in (scale.reshape(1, -1), shift.reshape(1, -1)):
            in_arrays.append(a); in_specs.append(full(a))
    if nf:
        for a in fuse_W_parts:
            in_arrays.append(a); in_specs.append(full(a))
    in_arrays.append(deg8); in_specs.append(row(deg8.shape[1]))

    if act == "lsm40":
        out_shape = [jax.ShapeDtypeStruct((N, 40), jnp.float32)]
        out_specs = [row(40)]
    else:
        out_shape = [jax.ShapeDtypeStruct((N, dout), jnp.float32)]
        out_specs = [row(dout)]
    if emit_bf:
        out_shape.append(jax.ShapeDtypeStruct((N, dout), jnp.bfloat16))
        out_specs.append(row(dout))
    if nf:
        out_shape.append(jax.ShapeDtypeStruct(
            (N, dnext), jnp.bfloat16 if fuse_bf else jnp.float32))
        out_specs.append(row(dnext))

    outs = pl.pallas_call(
        body,
        grid=(GRID,),
        in_specs=in_specs,
        out_specs=out_specs,
        out_shape=out_shape,
    )(*in_arrays)
    return outs


def _split(W):
    return [W[k * 256:(k + 1) * 256, :] for k in range(W.shape[0] // 256)]


def kernel(x, edge_index, params, bns):
    src = edge_index[0].astype(jnp.int32)
    dst = edge_index[1].astype(jnp.int32)

    order = jnp.argsort(dst)
    ds_ = dst[order]
    ss_ = src[order]
    bk = ds_ // NB_N
    bnd = jnp.searchsorted(
        ds_, jnp.arange(NBLK, dtype=jnp.int32) * NB_N).astype(jnp.int32)
    slot = bk * CE + jnp.arange(E, dtype=jnp.int32) - bnd[bk]
    srcp = jnp.zeros((SLOT,), jnp.int32).at[slot].set(ss_)
    srcp_a = srcp.reshape(NW, _SBT256, _EB256)
    srcp_b = srcp.reshape(NW, _SBT512, _EB512)
    dstl = jnp.full((SLOT,), NB_N, jnp.int32).at[slot].set(ds_ - bk * NB_N)
    dstl3 = dstl.reshape(NBLK, 1, CE)

    def agg(tabb, emit_deg=False):
        packed = lax.bitcast_convert_type(
            tabb.reshape(N, tabb.shape[1] // 2, 2), jnp.int32)
        if tabb.shape[1] == 512:
            msg = _gather512(packed, srcp_b)
        else:
            msg = _gather256(packed, srcp_a)
        msgb = lax.bitcast_convert_type(msg, jnp.bfloat16)
        msgb = msgb.reshape(SLOT, tabb.shape[1])
        return _segsum(msgb, dstl3, emit_deg=emit_deg)

    sqi = 1.0 / jnp.sqrt(1.0 + 1e-5)
    scales = [g * sqi for (g, _) in bns]
    shifts = [b for (_, b) in bns]

    (Wl1, b1, Wr1), (Wl2, b2, Wr2), (Wl3, b3, Wr3), (Wl4, b4, Wr4), \
        (Wl5, b5, Wr5), (Wl6, b6, Wr6), (Wl7, b7, Wr7) = params
    Wl7p = jnp.pad(Wl7, ((0, 0), (0, 24)))
    Wr7p = jnp.pad(Wr7, ((0, 0), (0, 24)))
    b7p = jnp.pad(b7, (0, 24))

    s1, deg8 = agg(x.astype(jnp.bfloat16), emit_deg=True)
    h1, h1b = _tc_layer([(s1, 0)], [(x, 0)], _split(Wl1), _split(Wr1), b1,
                        deg8, "relu", emit_bf=True)
    (s2,) = agg(h1b)
    g2, g2b = _tc_layer([(s2, 0), (s2, 1)], [(h1, 0), (h1, 1)], _split(Wl2),
                        _split(Wr2), b2, deg8, "bnrelu", scales[0], shifts[0],
                        emit_bf=True)
    (s3,) = agg(g2b)
    h3, h3b = _tc_layer([(s3, 0), (s3, 1)], [(g2, 0), (g2, 1)], _split(Wl3),
                        _split(Wr3), b3, deg8, "relu", emit_bf=True)
    (s4,) = agg(h3b)
    g4, y5b = _tc_layer([(s4, 0), (s4, 1)], [(h3, 0), (h3, 1)], _split(Wl4),
                        _split(Wr4), b4, deg8, "bnrelu", scales[1], shifts[1],
                        fuse_W_parts=_split(Wl5), fuse_bf=True)
    (s5,) = agg(y5b)
    h5, h5b = _tc_layer([(s5, 0)], [(g4, 0), (g4, 1)], None, _split(Wr5), b5,
                        deg8, "relu", s_pre=False, emit_bf=True)
    (s6,) = agg(h5b)
    g6, g6b = _tc_layer([(s6, 0)], [(h5, 0)], _split(Wl6), _split(Wr6), b6,
                        deg8, "bnrelu", scales[2], shifts[2], emit_bf=True)
    (s7,) = agg(g6b)
    (out,) = _tc_layer([(s7, 0)], [(g6, 0)], [Wl7p], [Wr7p], b7p, deg8,
                       "lsm40")
    return out


# speedup vs baseline: 1.6475x; 1.6475x over previous
import functools

import jax
import jax.numpy as jnp
from jax import lax
from jax.experimental import pallas as pl
from jax.experimental.pallas import tpu as pltpu
from jax.experimental.pallas import tpu_sc as plsc

N = 10000
E = 160000
EB = 128
NW = 32
NB_N = 256
NBLK = 40
CE = 4608
SLOT = NBLK * CE
SPT = SLOT // NW
SBT = SPT // EB
NS = NBLK * NB_N
M_BLK = 1000
GRID = N // M_BLK

_MESH = plsc.VectorSubcoreMesh(
    core_axis_name="c", subcore_axis_name="s", num_cores=2, num_subcores=16
)


_DEPTH = 6


def _make_gather(width):
    eb = 64 if width <= 256 else 32
    spt = SPT // 2
    sbt = spt // eb
    D = _DEPTH

    @functools.partial(
        pl.kernel,
        out_type=jax.ShapeDtypeStruct((SLOT // 2, width), jnp.float32),
        mesh=_MESH,
        scratch_types=[
            pltpu.VMEM((sbt, eb), jnp.int32),
            pltpu.VMEM((D, eb, width), jnp.float32),
            pltpu.SemaphoreType.DMA((D,)),
            pltpu.SemaphoreType.DMA((D,)),
        ],
    )
    def gather(table, srcs, msg, idxb, rows, gsem, wsem):
        c = lax.axis_index("c")
        s = lax.axis_index("s")
        w = s * 2 + c
        pltpu.sync_copy(srcs.at[w], idxb)

        def gstart(k, b):
            pltpu.async_copy(table.at[idxb.at[k]], rows.at[b], gsem.at[b])

        def gwait(k, b):
            pltpu.make_async_copy(table.at[idxb.at[k]], rows.at[b],
                                  gsem.at[b]).wait()

        def wstart(k, b):
            pltpu.async_copy(rows.at[b], msg.at[pl.ds(w * spt + k * eb, eb)],
                             wsem.at[b])

        def wwait(k, b):
            pltpu.make_async_copy(rows.at[b],
                                  msg.at[pl.ds(w * spt + k * eb, eb)],
                                  wsem.at[b]).wait()

        for j in range(D - 1):
            gstart(j, j)

        @pl.loop(0, sbt)
        def _(k):
            b = lax.rem(k, D)
            gwait(k, b)
            wstart(k, b)
            kn = k + D - 1

            @pl.when(kn < sbt)
            def _():
                bn = lax.rem(kn, D)

                @pl.when(k >= 1)
                def _():
                    wwait(k - 1, bn)

                gstart(kn, bn)

        for t in range(D):
            kk = sbt - 1 - t
            wwait(kk, lax.rem(kk, D))

    return gather, eb, sbt


_gather256, _EB256, _SBT256 = _make_gather(256)
_gather512, _EB512, _SBT512 = _make_gather(512)


def _segsum(msg, dstl3, emit_deg=False):
    width = msg.shape[1]
    nblk_h = NBLK // 2
    ns_h = nblk_h * NB_N

    def body(*refs):
        dl_ref, msg_ref = refs[0], refs[1]
        s_ref = refs[2]
        dl = dl_ref[0, 0, :]
        oh = jnp.where(
            lax.broadcasted_iota(jnp.int32, (NB_N, CE), 0) == dl[None, :],
            1.0, 0.0)
        s_ref[...] = jnp.dot(oh, msg_ref[...],
                             preferred_element_type=jnp.float32)
        if emit_deg:
            refs[3][...] = jnp.broadcast_to(
                jnp.sum(oh, axis=1, keepdims=True), (NB_N, 8))

    in_specs = [
        pl.BlockSpec((1, 1, CE), lambda i: (i, 0, 0)),
        pl.BlockSpec((CE, width), lambda i: (i, 0)),
    ]
    out_shape = [jax.ShapeDtypeStruct((ns_h, width), jnp.float32)]
    out_specs = [pl.BlockSpec((NB_N, width), lambda i: (i, 0))]
    if emit_deg:
        out_shape.append(jax.ShapeDtypeStruct((ns_h, 8), jnp.float32))
        out_specs.append(pl.BlockSpec((NB_N, 8), lambda i: (i, 0)))
    return pl.pallas_call(
        body, grid=(nblk_h,), in_specs=in_specs,
        out_specs=out_specs, out_shape=out_shape,
    )(dstl3, msg)


def _tc_layer(s_parts, x_parts, Wl_parts, Wr_parts, b, deg8, act,
              scale=None, shift=None, s_pre=True, fuse_W_parts=None,
              emit_bf=False, fuse_bf=False):
    ns, nx = len(s_parts), len(x_parts)
    dout = Wr_parts[0].shape[1]
    nW = len(Wl_parts) if s_pre else 0
    nf = len(fuse_W_parts) if fuse_W_parts is not None else 0
    dnext = fuse_W_parts[0].shape[1] if nf else 0
    has_bn = scale is not None

    def body(*refs):
        i = 0
        s_r = refs[i:i + ns]; i += ns
        x_r = refs[i:i + nx]; i += nx
        wl_r = refs[i:i + nW]; i += nW
        wr_r = refs[i:i + nx]; i += nx
        b_r = refs[i]; i += 1
        if has_bn:
            sc_r = refs[i]; i += 1
            sh_r = refs[i]; i += 1
        fw_r = refs[i:i + nf]; i += nf
        deg_r = refs[i]; i += 1
        out_r = refs[i]; i += 1
        bf_r = None
        if emit_bf:
            bf_r = refs[i]; i += 1
        y_r = refs[i] if nf else None

        invd = 1.0 / jnp.maximum(deg_r[...][:, 0:1], 1.0)
        z = b_r[...]
        if s_pre:
            for k in range(ns):
                z = z + jnp.dot(s_r[k][...] * invd, wl_r[k][...],
                                preferred_element_type=jnp.float32)
        else:
            z = z + s_r[0][...] * invd
        for k in range(nx):
            z = z + jnp.dot(x_r[k][...], wr_r[k][...],
                            preferred_element_type=jnp.float32)

        if act == "relu":
            h = jnp.maximum(z, 0.0)
        elif act == "bnrelu":
            h = jnp.maximum(z, 0.0) * sc_r[...] + sh_r[...]
        elif act == "lsm40":
            zm = jnp.where(lax.broadcasted_iota(jnp.int32, z.shape, 1) >= 40,
                           -1e30, z)
            m = jnp.max(zm, axis=1, keepdims=True)
            lse = m + jnp.log(jnp.sum(jnp.exp(zm - m), axis=1, keepdims=True))
            h = zm - lse
        else:
            h = z

        if act == "lsm40":
            out_r[...] = h[:, :40]
        else:
            out_r[...] = h
        if emit_bf:
            bf_r[...] = h.astype(jnp.bfloat16)
        if nf:
            y = jnp.dot(h[:, 0:256], fw_r[0][...],
                        preferred_element_type=jnp.float32)
            for k in range(1, nf):
                y = y + jnp.dot(h[:, k * 256:(k + 1) * 256], fw_r[k][...],
                                preferred_element_type=jnp.float32)
            y_r[...] = y.astype(jnp.bfloat16) if fuse_bf else y

    def pspec(part):
        a, cb = part
        wpart = min(256, a.shape[1])
        return a, pl.BlockSpec((M_BLK, wpart), lambda i, cb=cb: (i, cb))

    row = lambda shp: pl.BlockSpec((M_BLK, shp), lambda i: (i, 0))
    full = lambda a: pl.BlockSpec(a.shape, lambda i: (0, 0))

    in_arrays, in_specs = [], []
    for p in s_parts:
        a, sp = pspec(p); in_arrays.append(a); in_specs.append(sp)
    for p in x_parts:
        a, sp = pspec(p); in_arrays.append(a); in_specs.append(sp)
    if s_pre:
        for a in Wl_parts:
            in_arrays.append(a); in_specs.append(full(a))
    for a in Wr_parts:
        in_arrays.append(a); in_specs.append(full(a))
    b2 = b.reshape(1, -1)
    in_arrays.append(b2); in_specs.append(full(b2))
    if has_bn:
        for a in (scale.reshape(1, -1), shift.reshape(1, -1)):
            in_arrays.append(a); in_specs.append(full(a))
    if nf:
        for a in fuse_W_parts:
            in_arrays.append(a); in_specs.append(full(a))
    in_arrays.append(deg8); in_specs.append(row(deg8.shape[1]))

    if act == "lsm40":
        out_shape = [jax.ShapeDtypeStruct((N, 40), jnp.float32)]
        out_specs = [row(40)]
    else:
        out_shape = [jax.ShapeDtypeStruct((N, dout), jnp.float32)]
        out_specs = [row(dout)]
    if emit_bf:
        out_shape.append(jax.ShapeDtypeStruct((N, dout), jnp.bfloat16))
        out_specs.append(row(dout))
    if nf:
        out_shape.append(jax.ShapeDtypeStruct(
            (N, dnext), jnp.bfloat16 if fuse_bf else jnp.float32))
        out_specs.append(row(dnext))

    outs = pl.pallas_call(
        body,
        grid=(GRID,),
        in_specs=in_specs,
        out_specs=out_specs,
        out_shape=out_shape,
    )(*in_arrays)
    return outs


def _split(W):
    return [W[k * 256:(k + 1) * 256, :] for k in range(W.shape[0] // 256)]


def kernel(x, edge_index, params, bns):
    src = edge_index[0].astype(jnp.int32)
    dst = edge_index[1].astype(jnp.int32)

    order = jnp.argsort(dst)
    ds_ = dst[order]
    ss_ = src[order]
    bk = ds_ // NB_N
    bnd = jnp.searchsorted(
        ds_, jnp.arange(NBLK, dtype=jnp.int32) * NB_N).astype(jnp.int32)
    slot = bk * CE + jnp.arange(E, dtype=jnp.int32) - bnd[bk]
    srcp = jnp.zeros((SLOT,), jnp.int32).at[slot].set(ss_)
    srcp_a = [srcp[h * SLOT // 2:(h + 1) * SLOT // 2]
              .reshape(NW, _SBT256, _EB256) for h in range(2)]
    srcp_b = [srcp[h * SLOT // 2:(h + 1) * SLOT // 2]
              .reshape(NW, _SBT512, _EB512) for h in range(2)]
    dstl = jnp.full((SLOT,), NB_N, jnp.int32).at[slot].set(ds_ - bk * NB_N)
    dstl3 = [dstl[h * SLOT // 2:(h + 1) * SLOT // 2]
             .reshape(NBLK // 2, 1, CE) for h in range(2)]

    def agg(tab, emit_deg=False):
        gat, sp = ((_gather512, srcp_b) if tab.shape[1] == 512
                   else (_gather256, srcp_a))
        msgs = [gat(tab, sp[0]), gat(tab, sp[1])]
        res = [_segsum(msgs[h], dstl3[h], emit_deg=emit_deg)
               for h in range(2)]
        return [jnp.concatenate(p) for p in zip(*res)]

    sqi = 1.0 / jnp.sqrt(1.0 + 1e-5)
    scales = [g * sqi for (g, _) in bns]
    shifts = [b for (_, b) in bns]

    (Wl1, b1, Wr1), (Wl2, b2, Wr2), (Wl3, b3, Wr3), (Wl4, b4, Wr4), \
        (Wl5, b5, Wr5), (Wl6, b6, Wr6), (Wl7, b7, Wr7) = params
    Wl7p = jnp.pad(Wl7, ((0, 0), (0, 24)))
    Wr7p = jnp.pad(Wr7, ((0, 0), (0, 24)))
    b7p = jnp.pad(b7, (0, 24))

    s1, deg8 = agg(x, emit_deg=True)
    (h1,) = _tc_layer([(s1, 0)], [(x, 0)], _split(Wl1), _split(Wr1), b1,
                      deg8, "relu")
    (s2,) = agg(h1)
    (g2,) = _tc_layer([(s2, 0), (s2, 1)], [(h1, 0), (h1, 1)], _split(Wl2),
                      _split(Wr2), b2, deg8, "bnrelu", scales[0], shifts[0])
    (s3,) = agg(g2)
    (h3,) = _tc_layer([(s3, 0), (s3, 1)], [(g2, 0), (g2, 1)], _split(Wl3),
                      _split(Wr3), b3, deg8, "relu")
    (s4,) = agg(h3)
    g4, y5 = _tc_layer([(s4, 0), (s4, 1)], [(h3, 0), (h3, 1)], _split(Wl4),
                       _split(Wr4), b4, deg8, "bnrelu", scales[1], shifts[1],
                       fuse_W_parts=_split(Wl5))
    (s5,) = agg(y5)
    (h5,) = _tc_layer([(s5, 0)], [(g4, 0), (g4, 1)], None, _split(Wr5), b5,
                      deg8, "relu", s_pre=False)
    (s6,) = agg(h5)
    (g6,) = _tc_layer([(s6, 0)], [(h5, 0)], _split(Wl6), _split(Wr6), b6,
                      deg8, "bnrelu", scales[2], shifts[2])
    (s7,) = agg(g6)
    (out,) = _tc_layer([(s7, 0)], [(g6, 0)], [Wl7p], [Wr7p], b7p, deg8,
                       "lsm40")
    return out


# speedup vs baseline: 1.6543x; 1.0041x over previous
import functools

import jax
import jax.numpy as jnp
from jax import lax
from jax.experimental import pallas as pl
from jax.experimental.pallas import tpu as pltpu
from jax.experimental.pallas import tpu_sc as plsc

N = 10000
E = 160000
EB = 128
NW = 32
NB_N = 256
NBLK = 40
CE = 4608
SLOT = NBLK * CE
SPT = SLOT // NW
SBT = SPT // EB
NS = NBLK * NB_N
M_BLK = 1000
GRID = N // M_BLK

_MESH = plsc.VectorSubcoreMesh(
    core_axis_name="c", subcore_axis_name="s", num_cores=2, num_subcores=16
)


_DEPTH = 6


def _make_gather(width):
    eb = 64 if width <= 256 else 32
    spt = SPT // 2
    sbt = spt // eb
    D = _DEPTH

    @functools.partial(
        pl.kernel,
        out_type=jax.ShapeDtypeStruct((SLOT // 2, width), jnp.float32),
        mesh=_MESH,
        scratch_types=[
            pltpu.VMEM((sbt, eb), jnp.int32),
            pltpu.VMEM((D, eb, width), jnp.float32),
            pltpu.SemaphoreType.DMA((D,)),
            pltpu.SemaphoreType.DMA((D,)),
        ],
    )
    def gather(table, srcs, msg, idxb, rows, gsem, wsem):
        c = lax.axis_index("c")
        s = lax.axis_index("s")
        w = s * 2 + c
        pltpu.sync_copy(srcs.at[w], idxb)

        def gstart(k, b):
            pltpu.async_copy(table.at[idxb.at[k]], rows.at[b], gsem.at[b])

        def gwait(k, b):
            pltpu.make_async_copy(table.at[idxb.at[k]], rows.at[b],
                                  gsem.at[b]).wait()

        def wstart(k, b):
            pltpu.async_copy(rows.at[b], msg.at[pl.ds(w * spt + k * eb, eb)],
                             wsem.at[b])

        def wwait(k, b):
            pltpu.make_async_copy(rows.at[b],
                                  msg.at[pl.ds(w * spt + k * eb, eb)],
                                  wsem.at[b]).wait()

        for j in range(D - 1):
            gstart(j, j)

        @pl.loop(0, sbt)
        def _(k):
            b = lax.rem(k, D)
            gwait(k, b)
            wstart(k, b)
            kn = k + D - 1

            @pl.when(kn < sbt)
            def _():
                bn = lax.rem(kn, D)

                @pl.when(k >= 1)
                def _():
                    wwait(k - 1, bn)

                gstart(kn, bn)

        for t in range(D):
            kk = sbt - 1 - t
            wwait(kk, lax.rem(kk, D))

    return gather, eb, sbt


_gather256, _EB256, _SBT256 = _make_gather(256)
_gather512, _EB512, _SBT512 = _make_gather(512)


def _segsum(msg, dstl3, emit_deg=False):
    width = msg.shape[1]
    nblk_h = NBLK // 2
    ns_h = nblk_h * NB_N

    def body(*refs):
        dl_ref, msg_ref = refs[0], refs[1]
        s_ref = refs[2]
        dl = dl_ref[0, 0, :]
        oh = jnp.where(
            lax.broadcasted_iota(jnp.int32, (NB_N, CE), 0) == dl[None, :],
            1.0, 0.0)
        s_ref[...] = jnp.dot(oh, msg_ref[...],
                             preferred_element_type=jnp.float32)
        if emit_deg:
            refs[3][...] = jnp.broadcast_to(
                jnp.sum(oh, axis=1, keepdims=True), (NB_N, 8))

    in_specs = [
        pl.BlockSpec((1, 1, CE), lambda i: (i, 0, 0)),
        pl.BlockSpec((CE, width), lambda i: (i, 0)),
    ]
    out_shape = [jax.ShapeDtypeStruct((ns_h, width), jnp.float32)]
    out_specs = [pl.BlockSpec((NB_N, width), lambda i: (i, 0))]
    if emit_deg:
        out_shape.append(jax.ShapeDtypeStruct((ns_h, 8), jnp.float32))
        out_specs.append(pl.BlockSpec((NB_N, 8), lambda i: (i, 0)))
    return pl.pallas_call(
        body, grid=(nblk_h,), in_specs=in_specs,
        out_specs=out_specs, out_shape=out_shape,
    )(dstl3, msg)


def _tc_layer(s_parts, x_parts, Wl_parts, Wr_parts, b, deg8, act,
              scale=None, shift=None, s_pre=True, fuse_W_parts=None,
              emit_bf=False, fuse_bf=False):
    ns, nx = len(s_parts), len(x_parts)
    dout = Wr_parts[0].shape[1]
    nW = len(Wl_parts) if s_pre else 0
    nf = len(fuse_W_parts) if fuse_W_parts is not None else 0
    dnext = fuse_W_parts[0].shape[1] if nf else 0
    has_bn = scale is not None

    def body(*refs):
        i = 0
        s_r = refs[i:i + ns]; i += ns
        x_r = refs[i:i + nx]; i += nx
        wl_r = refs[i:i + nW]; i += nW
        wr_r = refs[i:i + nx]; i += nx
        b_r = refs[i]; i += 1
        if has_bn:
            sc_r = refs[i]; i += 1
            sh_r = refs[i]; i += 1
        fw_r = refs[i:i + nf]; i += nf
        deg_r = refs[i]; i += 1
        out_r = refs[i]; i += 1
        bf_r = None
        if emit_bf:
            bf_r = refs[i]; i += 1
        y_r = refs[i] if nf else None

        invd = 1.0 / jnp.maximum(deg_r[...][:, 0:1], 1.0)
        z = b_r[...]
        if s_pre:
            for k in range(ns):
                z = z + jnp.dot(s_r[k][...] * invd, wl_r[k][...],
                                preferred_element_type=jnp.float32)
        else:
            z = z + s_r[0][...] * invd
        for k in range(nx):
            z = z + jnp.dot(x_r[k][...], wr_r[k][...],
                            preferred_element_type=jnp.float32)

        if act == "relu":
            h = jnp.maximum(z, 0.0)
        elif act == "bnrelu":
            h = jnp.maximum(z, 0.0) * sc_r[...] + sh_r[...]
        elif act == "lsm40":
            zm = jnp.where(lax.broadcasted_iota(jnp.int32, z.shape, 1) >= 40,
                           -1e30, z)
            m = jnp.max(zm, axis=1, keepdims=True)
            lse = m + jnp.log(jnp.sum(jnp.exp(zm - m), axis=1, keepdims=True))
            h = zm - lse
        else:
            h = z

        if act == "lsm40":
            out_r[...] = h[:, :40]
        else:
            out_r[...] = h
        if emit_bf:
            bf_r[...] = h.astype(jnp.bfloat16)
        if nf:
            y = jnp.dot(h[:, 0:256], fw_r[0][...],
                        preferred_element_type=jnp.float32)
            for k in range(1, nf):
                y = y + jnp.dot(h[:, k * 256:(k + 1) * 256], fw_r[k][...],
                                preferred_element_type=jnp.float32)
            y_r[...] = y.astype(jnp.bfloat16) if fuse_bf else y

    def pspec(part):
        a, cb = part
        wpart = min(256, a.shape[1])
        return a, pl.BlockSpec((M_BLK, wpart), lambda i, cb=cb: (i, cb))

    row = lambda shp: pl.BlockSpec((M_BLK, shp), lambda i: (i, 0))
    full = lambda a: pl.BlockSpec(a.shape, lambda i: (0, 0))

    in_arrays, in_specs = [], []
    for p in s_parts:
        a, sp = pspec(p); in_arrays.append(a); in_specs.append(sp)
    for p in x_parts:
        a, sp = pspec(p); in_arrays.append(a); in_specs.append(sp)
    if s_pre:
        for a in Wl_parts:
            in_arrays.append(a); in_specs.append(full(a))
    for a in Wr_parts:
        in_arrays.append(a); in_specs.append(full(a))
    b2 = b.reshape(1, -1)
    in_arrays.append(b2); in_specs.append(full(b2))
    if has_bn:
        for a in (scale.reshape(1, -1), shift.reshape(1, -1)):
            in_arrays.append(a); in_specs.append(full(a))
    if nf:
        for a in fuse_W_parts:
            in_arrays.append(a); in_specs.append(full(a))
    in_arrays.append(deg8); in_specs.append(row(deg8.shape[1]))

    if act == "lsm40":
        out_shape = [jax.ShapeDtypeStruct((N, 40), jnp.float32)]
        out_specs = [row(40)]
    else:
        out_shape = [jax.ShapeDtypeStruct((N, dout), jnp.float32)]
        out_specs = [row(dout)]
    if emit_bf:
        out_shape.append(jax.ShapeDtypeStruct((N, dout), jnp.bfloat16))
        out_specs.append(row(dout))
    if nf:
        out_shape.append(jax.ShapeDtypeStruct(
            (N, dnext), jnp.bfloat16 if fuse_bf else jnp.float32))
        out_specs.append(row(dnext))

    outs = pl.pallas_call(
        body,
        grid=(GRID,),
        in_specs=in_specs,
        out_specs=out_specs,
        out_shape=out_shape,
    )(*in_arrays)
    return outs


def _split(W):
    return [W[k * 256:(k + 1) * 256, :] for k in range(W.shape[0] // 256)]


def kernel(x, edge_index, params, bns):
    src = edge_index[0].astype(jnp.int32)
    dst = edge_index[1].astype(jnp.int32)

    order = jnp.argsort((dst // NB_N) * (1 << 18) + src)
    ds_ = dst[order]
    ss_ = src[order]
    bk = ds_ // NB_N
    bnd = jnp.searchsorted(
        bk, jnp.arange(NBLK, dtype=jnp.int32)).astype(jnp.int32)
    slot = bk * CE + jnp.arange(E, dtype=jnp.int32) - bnd[bk]
    srcp = jnp.zeros((SLOT,), jnp.int32).at[slot].set(ss_)
    srcp_a = [srcp[h * SLOT // 2:(h + 1) * SLOT // 2]
              .reshape(NW, _SBT256, _EB256) for h in range(2)]
    srcp_b = [srcp[h * SLOT // 2:(h + 1) * SLOT // 2]
              .reshape(NW, _SBT512, _EB512) for h in range(2)]
    dstl = jnp.full((SLOT,), NB_N, jnp.int32).at[slot].set(ds_ - bk * NB_N)
    dstl3 = [dstl[h * SLOT // 2:(h + 1) * SLOT // 2]
             .reshape(NBLK // 2, 1, CE) for h in range(2)]

    def agg(tab, emit_deg=False):
        gat, sp = ((_gather512, srcp_b) if tab.shape[1] == 512
                   else (_gather256, srcp_a))
        msgs = [gat(tab, sp[0]), gat(tab, sp[1])]
        res = [_segsum(msgs[h], dstl3[h], emit_deg=emit_deg)
               for h in range(2)]
        return [jnp.concatenate(p) for p in zip(*res)]

    sqi = 1.0 / jnp.sqrt(1.0 + 1e-5)
    scales = [g * sqi for (g, _) in bns]
    shifts = [b for (_, b) in bns]

    (Wl1, b1, Wr1), (Wl2, b2, Wr2), (Wl3, b3, Wr3), (Wl4, b4, Wr4), \
        (Wl5, b5, Wr5), (Wl6, b6, Wr6), (Wl7, b7, Wr7) = params
    Wl7p = jnp.pad(Wl7, ((0, 0), (0, 24)))
    Wr7p = jnp.pad(Wr7, ((0, 0), (0, 24)))
    b7p = jnp.pad(b7, (0, 24))

    s1, deg8 = agg(x, emit_deg=True)
    (h1,) = _tc_layer([(s1, 0)], [(x, 0)], _split(Wl1), _split(Wr1), b1,
                      deg8, "relu")
    (s2,) = agg(h1)
    (g2,) = _tc_layer([(s2, 0), (s2, 1)], [(h1, 0), (h1, 1)], _split(Wl2),
                      _split(Wr2), b2, deg8, "bnrelu", scales[0], shifts[0])
    (s3,) = agg(g2)
    (h3,) = _tc_layer([(s3, 0), (s3, 1)], [(g2, 0), (g2, 1)], _split(Wl3),
                      _split(Wr3), b3, deg8, "relu")
    (s4,) = agg(h3)
    g4, y5 = _tc_layer([(s4, 0), (s4, 1)], [(h3, 0), (h3, 1)], _split(Wl4),
                       _split(Wr4), b4, deg8, "bnrelu", scales[1], shifts[1],
                       fuse_W_parts=_split(Wl5))
    (s5,) = agg(y5)
    (h5,) = _tc_layer([(s5, 0)], [(g4, 0), (g4, 1)], None, _split(Wr5), b5,
                      deg8, "relu", s_pre=False)
    (s6,) = agg(h5)
    (g6,) = _tc_layer([(s6, 0)], [(h5, 0)], _split(Wl6), _split(Wr6), b6,
                      deg8, "bnrelu", scales[2], shifts[2])
    (s7,) = agg(g6)
    (out,) = _tc_layer([(s7, 0)], [(g6, 0)], [Wl7p], [Wr7p], b7p, deg8,
                       "lsm40")
    return out
